# Initial kernel scaffold; baseline (speedup 1.0000x reference)
import jax
import jax.numpy as jnp
from jax.experimental import pallas as pl


def kernel(feature, edge_index, labels, idx_train, Wd0, bd0, Wd1, bd1, Wd2, bd2, Wd3, bd3, p0, p1, p2, Wu0, bu0, Wu1, bu1, Wu2, bu2):
    raise NotImplementedError("write your pallas kernel here")


# speedup vs baseline: 1.3054x; 1.3054x over previous
import functools
import math

import jax
import jax.numpy as jnp
from jax.experimental import pallas as pl
from jax.experimental.pallas import tpu as pltpu

N = 10000
NFEAT = 128
NHID = 128
NCLASS = 64
DEPTH = 3
RATIO = 0.5
TEMP = 0.5
WEIGHT_CPC = 0.5
SUB = 2048


def _mm_body(x_ref, w_ref, b_ref, o_ref, *, relu):
    acc = jnp.dot(x_ref[...], w_ref[...], preferred_element_type=jnp.float32)
    acc = acc + b_ref[...][None, :]
    if relu:
        acc = jnp.maximum(acc, 0.0)
    o_ref[...] = acc


def matmul_bias(x, w, b, relu=False, block=512):
    n, k = x.shape
    ko, m = w.shape
    grid = (pl.cdiv(n, block),)
    return pl.pallas_call(
        functools.partial(_mm_body, relu=relu),
        grid=grid,
        in_specs=[
            pl.BlockSpec((block, k), lambda i: (i, 0)),
            pl.BlockSpec((ko, m), lambda i: (0, 0)),
            pl.BlockSpec((m,), lambda i: (0,)),
        ],
        out_specs=pl.BlockSpec((block, m), lambda i: (i, 0)),
        out_shape=jax.ShapeDtypeStruct((n, m), jnp.float32),
    )(x, w, b)


def _gcn(x, src, dst, ew, W, b, n, relu):
    deg = jnp.zeros((n,), x.dtype).at[dst].add(ew) + 1.0
    dis = jax.lax.rsqrt(deg)
    norm = ew * dis[src] * dis[dst]
    xw = matmul_bias(x, W, jnp.zeros((W.shape[1],), x.dtype))
    msg = xw[src] * norm[:, None]
    out = jnp.zeros((n, W.shape[1]), x.dtype).at[dst].add(msg)
    out = out + xw * (dis * dis)[:, None] + b
    if relu:
        out = jnp.maximum(out, 0.0)
    return out


def _pool(x, src, dst, ew, p, k, n):
    score = (x @ p) / (jnp.linalg.norm(p) + 1e-12)
    vals, perm = jax.lax.top_k(score, k)
    x2 = x[perm] * jnp.tanh(vals)[:, None]
    keep = jnp.zeros((n,), dtype=bool).at[perm].set(True)
    newidx = jnp.zeros((n,), src.dtype).at[perm].set(jnp.arange(k, dtype=src.dtype))
    valid = keep[src] & keep[dst]
    src2 = jnp.where(valid, newidx[src], 0)
    dst2 = jnp.where(valid, newidx[dst], 0)
    ew2 = ew * valid.astype(x.dtype)
    return x2, src2, dst2, ew2, perm


def _supcon(feat, labels, temp):
    f = feat / (jnp.linalg.norm(feat, axis=1, keepdims=True) + 1e-12)
    sim = f @ f.T / temp
    m = feat.shape[0]
    eye = jnp.eye(m, dtype=bool)
    logits = sim - jax.lax.stop_gradient(jnp.max(sim, axis=1, keepdims=True))
    expl = jnp.exp(logits) * (~eye)
    logprob = logits - jnp.log(jnp.sum(expl, axis=1, keepdims=True) + 1e-12)
    pos = (labels[:, None] == labels[None, :]) & (~eye)
    cnt = jnp.sum(pos, axis=1)
    mlpp = jnp.sum(jnp.where(pos, logprob, 0.0), axis=1) / jnp.maximum(cnt, 1)
    return -jnp.mean(jnp.where(cnt > 0, mlpp, 0.0))


def kernel(feature, edge_index, labels, idx_train, Wd0, bd0, Wd1, bd1, Wd2, bd2, Wd3, bd3, p0, p1, p2, Wu0, bu0, Wu1, bu1, Wu2, bu2):
    Wd = [Wd0, Wd1, Wd2, Wd3]; bd = [bd0, bd1, bd2, bd3]
    ps = [p0, p1, p2]
    Wu = [Wu0, Wu1, Wu2]; bu = [bu0, bu1, bu2]

    n = feature.shape[0]
    src, dst = edge_index[0], edge_index[1]
    ew = jnp.ones((src.shape[0],), feature.dtype)
    x = _gcn(feature, src, dst, ew, Wd[0], bd[0], n, relu=True)
    xs = [x]; srcs = [src]; dsts = [dst]; ews = [ew]; ns = [n]; perms = []
    for i in range(1, DEPTH + 1):
        k = int(math.ceil(RATIO * n))
        x, src, dst, ew, perm = _pool(x, src, dst, ew, ps[i - 1], k, n)
        n = k
        x = _gcn(x, src, dst, ew, Wd[i], bd[i], n, relu=True)
        if i < DEPTH:
            xs.append(x); srcs.append(src); dsts.append(dst); ews.append(ew); ns.append(n)
        perms.append(perm)
    for i in range(DEPTH):
        j = DEPTH - 1 - i
        res = xs[j]
        up = jnp.zeros_like(res).at[perms[j]].set(x)
        x = res + up
        x = _gcn(x, srcs[j], dsts[j], ews[j], Wu[i], bu[i], ns[j],
                 relu=(i < DEPTH - 1))

    logits = x[idx_train]
    lt = labels[idx_train]
    logp = jax.nn.log_softmax(logits, axis=1)
    ce = -jnp.mean(jnp.take_along_axis(logp, lt[:, None], axis=1))
    scl = _supcon(logits, lt, TEMP)
    bcl = 0.0
    for i in range(DEPTH - 1):
        X = xs[0]; Y = xs[i + 1]
        Xd = jax.lax.stop_gradient(X); Yd = jax.lax.stop_gradient(Y)
        d2 = jnp.sum(Xd * Xd, 1)[:, None] - 2.0 * (Xd @ Yd.T) + jnp.sum(Yd * Yd, 1)[None, :]
        idx = jnp.argmin(d2, axis=1)
        cl = perms[i][idx]
        bcl = bcl + _supcon(X[:SUB], cl[:SUB], TEMP)
    return ce + (scl + bcl) * WEIGHT_CPC
